# Initial kernel scaffold; baseline (speedup 1.0000x reference)
import jax
import jax.numpy as jnp
from jax.experimental import pallas as pl


def kernel(x, edge_index, edge_type, batch, W1, Root1, b1, g1, be1, W2, Root2, b2, g2, be2, W3, aq, ak, b3, g3, be3, Wl, bl):
    raise NotImplementedError("write your pallas kernel here")


# speedup vs baseline: 5.9422x; 5.9422x over previous
import functools

import jax
import jax.numpy as jnp
from jax import lax
from jax.experimental import pallas as pl
from jax.experimental.pallas import tpu as pltpu
from jax.experimental.pallas import tpu_sc as plsc

N = 10000
E = 320000
R = 8
F = 128
H = 128
C = 16
G = 64

NPAD = 10240
NW = 32
EPW = E // NW
CH = 80
NCHUNK = EPW // CH
RPS = NPAD // 16

_mesh = plsc.VectorSubcoreMesh(core_axis_name="c", subcore_axis_name="s")


def _splat(v16, t):
    return jnp.zeros((16,), v16.dtype) + v16[t]


def _zero_rows128(rows_v):
    z = jnp.zeros((16,), jnp.float32)
    for i in range(CH):
        for f in range(H // 16):
            rows_v[i, pl.ds(f * 16, 16)] = z


def _init_acc(rows_v, acc, s):
    for j in range(RPS // CH):
        pltpu.sync_copy(rows_v, acc.at[pl.ds(s * RPS + j * CH, CH)])


def _drain(acc, out_hbm, c, s):
    pltpu.sync_copy(acc.at[pl.ds(s * RPS, RPS)],
                    out_hbm.at[c].at[pl.ds(s * RPS, RPS)])


@functools.partial(
    pl.kernel,
    mesh=_mesh,
    out_type=jax.ShapeDtypeStruct((2, NPAD, H), jnp.float32),
    scratch_types=[
        pltpu.VMEM((CH,), jnp.int32),
        pltpu.VMEM((CH,), jnp.int32),
        pltpu.VMEM((CH, H), jnp.float32),
        pltpu.VMEM_SHARED((NPAD, H), jnp.float32),
        pltpu.SemaphoreType.DMA,
    ],
)
def _sc_cnt(eye_hbm, dst_hbm, et_hbm, out_hbm, dst_v, et_v, rows_v, acc, sem):
    c = lax.axis_index("c")
    s = lax.axis_index("s")
    wid = c * 16 + s
    _zero_rows128(rows_v)
    _init_acc(rows_v, acc, s)
    plsc.subcore_barrier()

    def chunk(ch, _):
        base = wid * EPW + ch * CH
        pltpu.sync_copy(dst_hbm.at[pl.ds(base, CH)], dst_v)
        pltpu.sync_copy(et_hbm.at[pl.ds(base, CH)], et_v)
        pltpu.async_copy(eye_hbm.at[et_v], rows_v, sem).wait()
        pltpu.sync_copy(rows_v, acc.at[dst_v], add=True)
        return 0

    lax.fori_loop(0, NCHUNK, chunk, 0)
    plsc.subcore_barrier()
    _drain(acc, out_hbm, c, s)


@functools.partial(
    pl.kernel,
    mesh=_mesh,
    out_type=jax.ShapeDtypeStruct((2, NPAD, H), jnp.float32),
    scratch_types=[
        pltpu.VMEM((CH,), jnp.int32),
        pltpu.VMEM((CH,), jnp.int32),
        pltpu.VMEM((CH,), jnp.int32),
        pltpu.VMEM((CH,), jnp.int32),
        pltpu.VMEM((CH,), jnp.int32),
        pltpu.VMEM((CH,), jnp.float32),
        pltpu.VMEM((CH, H), jnp.float32),
        pltpu.VMEM_SHARED((NPAD, H), jnp.float32),
        pltpu.SemaphoreType.DMA,
        pltpu.SemaphoreType.DMA,
    ],
)
def _sc_agg_rgcn(xall_hbm, wflat_hbm, src_hbm, dst_hbm, et_hbm, out_hbm,
                 src_v, dst_v, et_v, idx_v, widx_v, w_v, rows_v, acc,
                 sem1, sem2):
    c = lax.axis_index("c")
    s = lax.axis_index("s")
    wid = c * 16 + s
    _zero_rows128(rows_v)
    _init_acc(rows_v, acc, s)
    plsc.subcore_barrier()

    def chunk(ch, _):
        base = wid * EPW + ch * CH
        pltpu.sync_copy(src_hbm.at[pl.ds(base, CH)], src_v)
        pltpu.sync_copy(dst_hbm.at[pl.ds(base, CH)], dst_v)
        pltpu.sync_copy(et_hbm.at[pl.ds(base, CH)], et_v)
        for j in range(CH // 16):
            sl = pl.ds(j * 16, 16)
            idx_v[sl] = et_v[sl] * N + src_v[sl]
            widx_v[sl] = dst_v[sl] * 16 + et_v[sl]
        cp1 = pltpu.async_copy(xall_hbm.at[idx_v], rows_v, sem1)
        cp2 = pltpu.async_copy(wflat_hbm.at[widx_v], w_v, sem2)
        cp2.wait()
        cp1.wait()
        for j in range(CH // 16):
            w16 = w_v[pl.ds(j * 16, 16)]
            for t in range(16):
                ws = _splat(w16, t)
                for f in range(H // 16):
                    sl = pl.ds(f * 16, 16)
                    rows_v[j * 16 + t, sl] = rows_v[j * 16 + t, sl] * ws
        pltpu.sync_copy(rows_v, acc.at[dst_v], add=True)
        return 0

    lax.fori_loop(0, NCHUNK, chunk, 0)
    plsc.subcore_barrier()
    _drain(acc, out_hbm, c, s)


@functools.partial(
    pl.kernel,
    mesh=_mesh,
    out_type=jax.ShapeDtypeStruct((E,), jnp.float32),
    scratch_types=[
        pltpu.VMEM((CH,), jnp.int32),
        pltpu.VMEM((CH,), jnp.int32),
        pltpu.VMEM((CH,), jnp.int32),
        pltpu.VMEM((CH,), jnp.int32),
        pltpu.VMEM((CH,), jnp.float32),
        pltpu.VMEM((CH,), jnp.float32),
        pltpu.VMEM((CH,), jnp.float32),
        pltpu.VMEM((CH,), jnp.float32),
        pltpu.SemaphoreType.DMA,
        pltpu.SemaphoreType.DMA,
        pltpu.SemaphoreType.DMA,
    ],
)
def _sc_logits_ex(qflat_hbm, kflat_hbm, bflat_hbm, src_hbm, dst_hbm, et_hbm,
                  ex_hbm,
                  src_v, dst_v, et_v, qidx_v, q_v, k_v, b_v, ex_v,
                  sem1, sem2, sem3):
    c = lax.axis_index("c")
    s = lax.axis_index("s")
    wid = c * 16 + s

    def chunk(ch, _):
        base = wid * EPW + ch * CH
        pltpu.sync_copy(src_hbm.at[pl.ds(base, CH)], src_v)
        pltpu.sync_copy(dst_hbm.at[pl.ds(base, CH)], dst_v)
        pltpu.sync_copy(et_hbm.at[pl.ds(base, CH)], et_v)
        for j in range(CH // 16):
            sl = pl.ds(j * 16, 16)
            qidx_v[sl] = dst_v[sl] * 16 + et_v[sl]
        cp1 = pltpu.async_copy(qflat_hbm.at[qidx_v], q_v, sem1)
        cp3 = pltpu.async_copy(bflat_hbm.at[dst_v], b_v, sem3)
        cp1.wait()
        for j in range(CH // 16):
            sl = pl.ds(j * 16, 16)
            qidx_v[sl] = src_v[sl] * 16 + et_v[sl]
        cp2 = pltpu.async_copy(kflat_hbm.at[qidx_v], k_v, sem2)
        cp2.wait()
        cp3.wait()
        for j in range(CH // 16):
            sl = pl.ds(j * 16, 16)
            l = q_v[sl] + k_v[sl]
            l = jnp.where(l >= 0.0, l, 0.2 * l)
            ex_v[sl] = jnp.exp(l - b_v[sl])
        pltpu.sync_copy(ex_v, ex_hbm.at[pl.ds(base, CH)])
        return 0

    lax.fori_loop(0, NCHUNK, chunk, 0)


@functools.partial(
    pl.kernel,
    mesh=_mesh,
    out_type=jax.ShapeDtypeStruct((2, NPAD, H), jnp.float32),
    scratch_types=[
        pltpu.VMEM((CH,), jnp.int32),
        pltpu.VMEM((CH,), jnp.int32),
        pltpu.VMEM((CH,), jnp.float32),
        pltpu.VMEM((CH, H), jnp.float32),
        pltpu.VMEM_SHARED((NPAD, H), jnp.float32),
        pltpu.SemaphoreType.DMA,
    ],
)
def _sc_exsum(ones_hbm, dst_hbm, et_hbm, ex_hbm, out_hbm,
              dst_v, et_v, ex_v, rows_v, acc, sem):
    c = lax.axis_index("c")
    s = lax.axis_index("s")
    wid = c * 16 + s
    _zero_rows128(rows_v)
    _init_acc(rows_v, acc, s)
    plsc.subcore_barrier()

    def chunk(ch, _):
        base = wid * EPW + ch * CH
        pltpu.sync_copy(dst_hbm.at[pl.ds(base, CH)], dst_v)
        pltpu.sync_copy(et_hbm.at[pl.ds(base, CH)], et_v)
        pltpu.sync_copy(ex_hbm.at[pl.ds(base, CH)], ex_v)
        pltpu.async_copy(ones_hbm.at[et_v], rows_v, sem).wait()
        for j in range(CH // 16):
            w16 = ex_v[pl.ds(j * 16, 16)]
            for t in range(16):
                ws = _splat(w16, t)
                for f in range(H // 16):
                    sl = pl.ds(f * 16, 16)
                    rows_v[j * 16 + t, sl] = rows_v[j * 16 + t, sl] * ws
        pltpu.sync_copy(rows_v, acc.at[dst_v], add=True)
        return 0

    lax.fori_loop(0, NCHUNK, chunk, 0)
    plsc.subcore_barrier()
    _drain(acc, out_hbm, c, s)


@functools.partial(
    pl.kernel,
    mesh=_mesh,
    out_type=jax.ShapeDtypeStruct((2, NPAD, H), jnp.float32),
    scratch_types=[
        pltpu.VMEM((CH,), jnp.int32),
        pltpu.VMEM((CH,), jnp.int32),
        pltpu.VMEM((CH,), jnp.int32),
        pltpu.VMEM((CH,), jnp.int32),
        pltpu.VMEM((CH,), jnp.float32),
        pltpu.VMEM((CH,), jnp.float32),
        pltpu.VMEM((CH, H), jnp.float32),
        pltpu.VMEM_SHARED((NPAD, H), jnp.float32),
        pltpu.SemaphoreType.DMA,
        pltpu.SemaphoreType.DMA,
    ],
)
def _sc_agg_rgat(xall_hbm, sinv_hbm, src_hbm, dst_hbm, et_hbm, ex_hbm,
                 out_hbm,
                 src_v, dst_v, et_v, idx_v, ex_v, sv_v, rows_v, acc,
                 sem1, sem2):
    c = lax.axis_index("c")
    s = lax.axis_index("s")
    wid = c * 16 + s
    _zero_rows128(rows_v)
    _init_acc(rows_v, acc, s)
    plsc.subcore_barrier()

    def chunk(ch, _):
        base = wid * EPW + ch * CH
        pltpu.sync_copy(src_hbm.at[pl.ds(base, CH)], src_v)
        pltpu.sync_copy(dst_hbm.at[pl.ds(base, CH)], dst_v)
        pltpu.sync_copy(et_hbm.at[pl.ds(base, CH)], et_v)
        pltpu.sync_copy(ex_hbm.at[pl.ds(base, CH)], ex_v)
        for j in range(CH // 16):
            sl = pl.ds(j * 16, 16)
            idx_v[sl] = et_v[sl] * N + src_v[sl]
        cp1 = pltpu.async_copy(xall_hbm.at[idx_v], rows_v, sem1)
        cp2 = pltpu.async_copy(sinv_hbm.at[dst_v], sv_v, sem2)
        cp2.wait()
        for j in range(CH // 16):
            sl = pl.ds(j * 16, 16)
            ex_v[sl] = ex_v[sl] * sv_v[sl]
        cp1.wait()
        for j in range(CH // 16):
            w16 = ex_v[pl.ds(j * 16, 16)]
            for t in range(16):
                ws = _splat(w16, t)
                for f in range(H // 16):
                    sl = pl.ds(f * 16, 16)
                    rows_v[j * 16 + t, sl] = rows_v[j * 16 + t, sl] * ws
        pltpu.sync_copy(rows_v, acc.at[dst_v], add=True)
        return 0

    lax.fori_loop(0, NCHUNK, chunk, 0)
    plsc.subcore_barrier()
    _drain(acc, out_hbm, c, s)


def _xall_body(x_ref, w_ref, o_ref):
    o_ref[0] = jnp.dot(x_ref[...], w_ref[0], preferred_element_type=jnp.float32)


def _xall(h, W):
    out = pl.pallas_call(
        _xall_body,
        grid=(R,),
        in_specs=[
            pl.BlockSpec((N, F), lambda r: (0, 0)),
            pl.BlockSpec((1, F, H), lambda r: (r, 0, 0)),
        ],
        out_specs=pl.BlockSpec((1, N, H), lambda r: (r, 0, 0)),
        out_shape=jax.ShapeDtypeStruct((R, N, H), jnp.float32),
    )(h, W)
    return out.reshape(R * N, H)


def _inv_body(p_ref, o_ref):
    cnt = p_ref[0, :, :16] + p_ref[1, :, :16]
    o_ref[...] = 1.0 / jnp.maximum(cnt, 1.0)


def _inv_table(parts):
    return pl.pallas_call(
        _inv_body,
        out_shape=jax.ShapeDtypeStruct((NPAD, 16), jnp.float32),
    )(parts)


def _epi_rgcn_body(p_ref, h_ref, root_ref, b_ref, g_ref, be_ref, o_ref):
    agg = p_ref[0, :N, :] + p_ref[1, :N, :]
    agg = agg + jnp.dot(h_ref[...], root_ref[...],
                        preferred_element_type=jnp.float32) + b_ref[...]
    mu = jnp.mean(agg, axis=0, keepdims=True)
    var = jnp.mean((agg - mu) ** 2, axis=0, keepdims=True)
    y = (agg - mu) / jnp.sqrt(var + 1e-5) * g_ref[...] + be_ref[...]
    o_ref[...] = jnp.maximum(y, 0.0)


def _epi_rgcn(parts, h, Root, b, g, be):
    return pl.pallas_call(
        _epi_rgcn_body,
        out_shape=jax.ShapeDtypeStruct((N, H), jnp.float32),
    )(parts, h, Root, b.reshape(1, H), g.reshape(1, H), be.reshape(1, H))


def _prep3_body(h_ref, w3_ref, aq_ref, ak_ref, q_ref, k_ref, b_ref):
    h = h_ref[...]
    vq = jnp.einsum("rhk,rk->rh", w3_ref[...], aq_ref[...],
                    preferred_element_type=jnp.float32)
    vk = jnp.einsum("rhk,rk->rh", w3_ref[...], ak_ref[...],
                    preferred_element_type=jnp.float32)
    pad = jnp.zeros((16 - R, H), jnp.float32)
    vq16 = jnp.concatenate([vq, pad], axis=0)
    vk16 = jnp.concatenate([vk, pad], axis=0)
    q = jnp.dot(h, vq16.T, preferred_element_type=jnp.float32)
    k = jnp.dot(h, vk16.T, preferred_element_type=jnp.float32)
    lane = lax.broadcasted_iota(jnp.int32, (1, 16), 1)
    valid = lane < R
    kmax = jnp.max(jnp.where(valid, k, -jnp.inf), axis=0, keepdims=True)
    bpre = jnp.max(jnp.where(valid, q + kmax, -jnp.inf), axis=1, keepdims=True)
    bnd = jnp.where(bpre >= 0.0, bpre, 0.2 * bpre)
    q_ref[...] = jnp.where(valid, q, 0.0)
    k_ref[...] = jnp.where(valid, k, 0.0)
    b_ref[...] = bnd


def _prep3(h, W3, aq, ak):
    return pl.pallas_call(
        _prep3_body,
        out_shape=(
            jax.ShapeDtypeStruct((N, 16), jnp.float32),
            jax.ShapeDtypeStruct((N, 16), jnp.float32),
            jax.ShapeDtypeStruct((N, 1), jnp.float32),
        ),
    )(h, W3, aq, ak)


def _sinv_body(p_ref, o_ref):
    stot = p_ref[0, :, :1] + p_ref[1, :, :1]
    o_ref[...] = 1.0 / jnp.maximum(stot, 1e-16)


def _sinv_table(parts):
    return pl.pallas_call(
        _sinv_body,
        out_shape=jax.ShapeDtypeStruct((NPAD, 1), jnp.float32),
    )(parts)


def _epi3_pool_body(p_ref, b_ref, g_ref, be_ref, batch_ref, wl_ref, bl_ref,
                    o_ref):
    agg = p_ref[0, :N, :] + p_ref[1, :N, :] + b_ref[...]
    mu = jnp.mean(agg, axis=0, keepdims=True)
    var = jnp.mean((agg - mu) ** 2, axis=0, keepdims=True)
    y = (agg - mu) / jnp.sqrt(var + 1e-5) * g_ref[...] + be_ref[...]
    h = jnp.maximum(y, 0.0)
    gid = lax.broadcasted_iota(jnp.int32, (G, N), 0)
    P = (batch_ref[...] == gid).astype(jnp.float32)
    cnt = jnp.sum(P, axis=1, keepdims=True)
    pooled = jnp.dot(P, h, preferred_element_type=jnp.float32)
    pooled = pooled / jnp.maximum(cnt, 1.0)
    o_ref[...] = jnp.dot(pooled, wl_ref[...],
                         preferred_element_type=jnp.float32) + bl_ref[...]


def _epi3_pool(parts, b3, g3, be3, batch, Wl, bl):
    return pl.pallas_call(
        _epi3_pool_body,
        out_shape=jax.ShapeDtypeStruct((G, C), jnp.float32),
    )(parts, b3.reshape(1, H), g3.reshape(1, H), be3.reshape(1, H),
      batch.reshape(1, N), Wl, bl.reshape(1, C))


def kernel(x, edge_index, edge_type, batch, W1, Root1, b1, g1, be1,
           W2, Root2, b2, g2, be2, W3, aq, ak, b3, g3, be3, Wl, bl):
    src = edge_index[0]
    dst = edge_index[1]
    et = edge_type
    eye128 = jnp.concatenate(
        [jnp.eye(16, dtype=jnp.float32),
         jnp.zeros((16, H - 16), jnp.float32)], axis=1)
    ones128 = jnp.ones((16, H), jnp.float32)

    cnt_parts = _sc_cnt(eye128, dst, et)
    wflat = _inv_table(cnt_parts).reshape(NPAD * 16)

    xall1 = _xall(x, W1)
    p1 = _sc_agg_rgcn(xall1, wflat, src, dst, et)
    h1 = _epi_rgcn(p1, x, Root1, b1, g1, be1)

    xall2 = _xall(h1, W2)
    p2 = _sc_agg_rgcn(xall2, wflat, src, dst, et)
    h2 = _epi_rgcn(p2, h1, Root2, b2, g2, be2)

    q16, k16, bnd = _prep3(h2, W3, aq, ak)
    qflat = q16.reshape(N * 16)
    kflat = k16.reshape(N * 16)
    bflat = bnd.reshape(N)
    ex = _sc_logits_ex(qflat, kflat, bflat, src, dst, et)
    s_parts = _sc_exsum(ones128, dst, et, ex)
    sinv = _sinv_table(s_parts).reshape(NPAD)
    xall3 = _xall(h2, W3)
    p3 = _sc_agg_rgat(xall3, sinv, src, dst, et, ex)
    return _epi3_pool(p3, b3, g3, be3, batch, Wl, bl)


# speedup vs baseline: 6.6768x; 1.1236x over previous
import functools

import jax
import jax.numpy as jnp
from jax import lax
from jax.experimental import pallas as pl
from jax.experimental.pallas import tpu as pltpu
from jax.experimental.pallas import tpu_sc as plsc

N = 10000
E = 320000
R = 8
F = 128
H = 128
C = 16
G = 64

NPAD = 10240
NW = 32
EPW = E // NW
CH = 80
NCHUNK = EPW // CH
RPS = NPAD // 16

_mesh = plsc.VectorSubcoreMesh(core_axis_name="c", subcore_axis_name="s")


def _splat(v16, t):
    return jnp.zeros((16,), v16.dtype) + v16[t]


def _zero_rows128(rows_v):
    z = jnp.zeros((16,), jnp.float32)
    for i in range(CH):
        for f in range(H // 16):
            rows_v[i, pl.ds(f * 16, 16)] = z


def _init_acc(rows_v, acc, s):
    for j in range(RPS // CH):
        pltpu.sync_copy(rows_v, acc.at[pl.ds(s * RPS + j * CH, CH)])


def _drain(acc, out_hbm, c, s):
    pltpu.sync_copy(acc.at[pl.ds(s * RPS, RPS)],
                    out_hbm.at[c].at[pl.ds(s * RPS, RPS)])


@functools.partial(
    pl.kernel,
    mesh=_mesh,
    out_type=jax.ShapeDtypeStruct((2, NPAD, H), jnp.float32),
    scratch_types=[
        pltpu.VMEM((CH,), jnp.int32),
        pltpu.VMEM((CH,), jnp.int32),
        pltpu.VMEM((CH, H), jnp.float32),
        pltpu.VMEM_SHARED((NPAD, H), jnp.float32),
        pltpu.SemaphoreType.DMA,
    ],
)
def _sc_cnt(eye_hbm, dst_hbm, et_hbm, out_hbm, dst_v, et_v, rows_v, acc, sem):
    c = lax.axis_index("c")
    s = lax.axis_index("s")
    wid = c * 16 + s
    _zero_rows128(rows_v)
    _init_acc(rows_v, acc, s)
    plsc.subcore_barrier()

    def chunk(ch, _):
        base = wid * EPW + ch * CH
        pltpu.sync_copy(dst_hbm.at[pl.ds(base, CH)], dst_v)
        pltpu.sync_copy(et_hbm.at[pl.ds(base, CH)], et_v)
        pltpu.async_copy(eye_hbm.at[et_v], rows_v, sem).wait()
        pltpu.sync_copy(rows_v, acc.at[dst_v], add=True)
        return 0

    lax.fori_loop(0, NCHUNK, chunk, 0)
    plsc.subcore_barrier()
    _drain(acc, out_hbm, c, s)


@functools.partial(
    pl.kernel,
    mesh=_mesh,
    out_type=jax.ShapeDtypeStruct((2, NPAD, H), jnp.float32),
    scratch_types=[
        pltpu.VMEM((CH,), jnp.int32), pltpu.VMEM((CH,), jnp.int32),
        pltpu.VMEM((CH,), jnp.int32), pltpu.VMEM((CH,), jnp.int32),
        pltpu.VMEM((CH,), jnp.int32), pltpu.VMEM((CH,), jnp.int32),
        pltpu.VMEM((CH,), jnp.int32), pltpu.VMEM((CH,), jnp.int32),
        pltpu.VMEM((CH,), jnp.int32), pltpu.VMEM((CH,), jnp.int32),
        pltpu.VMEM((CH,), jnp.float32), pltpu.VMEM((CH,), jnp.float32),
        pltpu.VMEM((CH, H), jnp.float32), pltpu.VMEM((CH, H), jnp.float32),
        pltpu.VMEM_SHARED((NPAD, H), jnp.float32),
        pltpu.SemaphoreType.DMA, pltpu.SemaphoreType.DMA,
        pltpu.SemaphoreType.DMA, pltpu.SemaphoreType.DMA,
    ],
)
def _sc_agg_rgcn(xall_hbm, wflat_hbm, src_hbm, dst_hbm, et_hbm, out_hbm,
                 src0, src1, dst0, dst1, et0, et1, idx0, idx1, wix0, wix1,
                 w0, w1, rows0, rows1, acc,
                 semr0, semw0, semr1, semw1):
    c = lax.axis_index("c")
    s = lax.axis_index("s")
    wid = c * 16 + s
    _zero_rows128(rows0)
    _init_acc(rows0, acc, s)
    plsc.subcore_barrier()

    bank = ((src0, dst0, et0, idx0, wix0, w0, rows0, semr0, semw0),
            (src1, dst1, et1, idx1, wix1, w1, rows1, semr1, semw1))

    def load_edges(ch, b):
        sv, dv, ev, iv, wv, wlv, rv, sr, sw = bank[b]
        base = wid * EPW + ch * CH
        pltpu.sync_copy(src_hbm.at[pl.ds(base, CH)], sv)
        pltpu.sync_copy(dst_hbm.at[pl.ds(base, CH)], dv)
        pltpu.sync_copy(et_hbm.at[pl.ds(base, CH)], ev)
        for j in range(CH // 16):
            sl = pl.ds(j * 16, 16)
            iv[sl] = ev[sl] * N + sv[sl]
            wv[sl] = dv[sl] * 16 + ev[sl]
        pltpu.async_copy(xall_hbm.at[iv], rv, sr)
        pltpu.async_copy(wflat_hbm.at[wv], wlv, sw)

    def process(b):
        sv, dv, ev, iv, wv, wlv, rv, sr, sw = bank[b]
        pltpu.make_async_copy(wflat_hbm.at[wv], wlv, sw).wait()
        pltpu.make_async_copy(xall_hbm.at[iv], rv, sr).wait()
        for j in range(CH // 16):
            w16 = wlv[pl.ds(j * 16, 16)]
            for t in range(16):
                ws = _splat(w16, t)
                for f in range(H // 16):
                    sl = pl.ds(f * 16, 16)
                    rv[j * 16 + t, sl] = rv[j * 16 + t, sl] * ws
        pltpu.sync_copy(rv, acc.at[dv], add=True)

    load_edges(0, 0)

    def chunk(g, _):
        load_edges(2 * g + 1, 1)
        process(0)
        load_edges(2 * g + 2, 0)
        process(1)
        return 0

    lax.fori_loop(0, (NCHUNK - 1) // 2, chunk, 0)
    process(0)
    plsc.subcore_barrier()
    _drain(acc, out_hbm, c, s)


@functools.partial(
    pl.kernel,
    mesh=_mesh,
    out_type=jax.ShapeDtypeStruct((E,), jnp.float32),
    scratch_types=[
        pltpu.VMEM((CH,), jnp.int32),
        pltpu.VMEM((CH,), jnp.int32),
        pltpu.VMEM((CH,), jnp.int32),
        pltpu.VMEM((CH,), jnp.int32),
        pltpu.VMEM((CH,), jnp.float32),
        pltpu.VMEM((CH,), jnp.float32),
        pltpu.VMEM((CH,), jnp.float32),
        pltpu.VMEM((CH,), jnp.float32),
        pltpu.SemaphoreType.DMA,
        pltpu.SemaphoreType.DMA,
        pltpu.SemaphoreType.DMA,
    ],
)
def _sc_logits_ex(qflat_hbm, kflat_hbm, bflat_hbm, src_hbm, dst_hbm, et_hbm,
                  ex_hbm,
                  src_v, dst_v, et_v, qidx_v, q_v, k_v, b_v, ex_v,
                  sem1, sem2, sem3):
    c = lax.axis_index("c")
    s = lax.axis_index("s")
    wid = c * 16 + s

    def chunk(ch, _):
        base = wid * EPW + ch * CH
        pltpu.sync_copy(src_hbm.at[pl.ds(base, CH)], src_v)
        pltpu.sync_copy(dst_hbm.at[pl.ds(base, CH)], dst_v)
        pltpu.sync_copy(et_hbm.at[pl.ds(base, CH)], et_v)
        for j in range(CH // 16):
            sl = pl.ds(j * 16, 16)
            qidx_v[sl] = dst_v[sl] * 16 + et_v[sl]
        cp1 = pltpu.async_copy(qflat_hbm.at[qidx_v], q_v, sem1)
        cp3 = pltpu.async_copy(bflat_hbm.at[dst_v], b_v, sem3)
        cp1.wait()
        for j in range(CH // 16):
            sl = pl.ds(j * 16, 16)
            qidx_v[sl] = src_v[sl] * 16 + et_v[sl]
        cp2 = pltpu.async_copy(kflat_hbm.at[qidx_v], k_v, sem2)
        cp2.wait()
        cp3.wait()
        for j in range(CH // 16):
            sl = pl.ds(j * 16, 16)
            l = q_v[sl] + k_v[sl]
            l = jnp.where(l >= 0.0, l, 0.2 * l)
            ex_v[sl] = jnp.exp(l - b_v[sl])
        pltpu.sync_copy(ex_v, ex_hbm.at[pl.ds(base, CH)])
        return 0

    lax.fori_loop(0, NCHUNK, chunk, 0)


@functools.partial(
    pl.kernel,
    mesh=_mesh,
    out_type=jax.ShapeDtypeStruct((2, NPAD, H), jnp.float32),
    scratch_types=[
        pltpu.VMEM((CH,), jnp.int32),
        pltpu.VMEM((CH,), jnp.int32),
        pltpu.VMEM((CH,), jnp.float32),
        pltpu.VMEM((CH, H), jnp.float32),
        pltpu.VMEM_SHARED((NPAD, H), jnp.float32),
        pltpu.SemaphoreType.DMA,
    ],
)
def _sc_exsum(ones_hbm, dst_hbm, et_hbm, ex_hbm, out_hbm,
              dst_v, et_v, ex_v, rows_v, acc, sem):
    c = lax.axis_index("c")
    s = lax.axis_index("s")
    wid = c * 16 + s
    _zero_rows128(rows_v)
    _init_acc(rows_v, acc, s)
    plsc.subcore_barrier()

    def chunk(ch, _):
        base = wid * EPW + ch * CH
        pltpu.sync_copy(dst_hbm.at[pl.ds(base, CH)], dst_v)
        pltpu.sync_copy(et_hbm.at[pl.ds(base, CH)], et_v)
        pltpu.sync_copy(ex_hbm.at[pl.ds(base, CH)], ex_v)
        pltpu.async_copy(ones_hbm.at[et_v], rows_v, sem).wait()
        for j in range(CH // 16):
            w16 = ex_v[pl.ds(j * 16, 16)]
            for t in range(16):
                ws = _splat(w16, t)
                for f in range(H // 16):
                    sl = pl.ds(f * 16, 16)
                    rows_v[j * 16 + t, sl] = rows_v[j * 16 + t, sl] * ws
        pltpu.sync_copy(rows_v, acc.at[dst_v], add=True)
        return 0

    lax.fori_loop(0, NCHUNK, chunk, 0)
    plsc.subcore_barrier()
    _drain(acc, out_hbm, c, s)


@functools.partial(
    pl.kernel,
    mesh=_mesh,
    out_type=jax.ShapeDtypeStruct((2, NPAD, H), jnp.float32),
    scratch_types=[
        pltpu.VMEM((CH,), jnp.int32),
        pltpu.VMEM((CH,), jnp.int32),
        pltpu.VMEM((CH,), jnp.int32),
        pltpu.VMEM((CH,), jnp.int32),
        pltpu.VMEM((CH,), jnp.float32),
        pltpu.VMEM((CH,), jnp.float32),
        pltpu.VMEM((CH, H), jnp.float32),
        pltpu.VMEM_SHARED((NPAD, H), jnp.float32),
        pltpu.SemaphoreType.DMA,
        pltpu.SemaphoreType.DMA,
    ],
)
def _sc_agg_rgat(xall_hbm, sinv_hbm, src_hbm, dst_hbm, et_hbm, ex_hbm,
                 out_hbm,
                 src_v, dst_v, et_v, idx_v, ex_v, sv_v, rows_v, acc,
                 sem1, sem2):
    c = lax.axis_index("c")
    s = lax.axis_index("s")
    wid = c * 16 + s
    _zero_rows128(rows_v)
    _init_acc(rows_v, acc, s)
    plsc.subcore_barrier()

    def chunk(ch, _):
        base = wid * EPW + ch * CH
        pltpu.sync_copy(src_hbm.at[pl.ds(base, CH)], src_v)
        pltpu.sync_copy(dst_hbm.at[pl.ds(base, CH)], dst_v)
        pltpu.sync_copy(et_hbm.at[pl.ds(base, CH)], et_v)
        pltpu.sync_copy(ex_hbm.at[pl.ds(base, CH)], ex_v)
        for j in range(CH // 16):
            sl = pl.ds(j * 16, 16)
            idx_v[sl] = et_v[sl] * N + src_v[sl]
        cp1 = pltpu.async_copy(xall_hbm.at[idx_v], rows_v, sem1)
        cp2 = pltpu.async_copy(sinv_hbm.at[dst_v], sv_v, sem2)
        cp2.wait()
        for j in range(CH // 16):
            sl = pl.ds(j * 16, 16)
            ex_v[sl] = ex_v[sl] * sv_v[sl]
        cp1.wait()
        for j in range(CH // 16):
            w16 = ex_v[pl.ds(j * 16, 16)]
            for t in range(16):
                ws = _splat(w16, t)
                for f in range(H // 16):
                    sl = pl.ds(f * 16, 16)
                    rows_v[j * 16 + t, sl] = rows_v[j * 16 + t, sl] * ws
        pltpu.sync_copy(rows_v, acc.at[dst_v], add=True)
        return 0

    lax.fori_loop(0, NCHUNK, chunk, 0)
    plsc.subcore_barrier()
    _drain(acc, out_hbm, c, s)


@functools.partial(
    pl.kernel,
    mesh=_mesh,
    out_type=(
        jax.ShapeDtypeStruct((E,), jnp.float32),
        jax.ShapeDtypeStruct((2, NPAD, H), jnp.float32),
    ),
    scratch_types=[
        pltpu.VMEM((CH,), jnp.int32),
        pltpu.VMEM((CH,), jnp.int32),
        pltpu.VMEM((CH,), jnp.int32),
        pltpu.VMEM((CH,), jnp.int32),
        pltpu.VMEM((CH,), jnp.float32),
        pltpu.VMEM((CH,), jnp.float32),
        pltpu.VMEM((CH,), jnp.float32),
        pltpu.VMEM((CH,), jnp.float32),
        pltpu.VMEM((CH, H), jnp.float32),
        pltpu.VMEM_SHARED((NPAD, H), jnp.float32),
        pltpu.SemaphoreType.DMA,
        pltpu.SemaphoreType.DMA,
        pltpu.SemaphoreType.DMA,
        pltpu.SemaphoreType.DMA,
    ],
)
def _sc_logits_fused(ones_hbm, qflat_hbm, kflat_hbm, bflat_hbm, src_hbm,
                     dst_hbm, et_hbm, ex_hbm, sp_hbm,
                     src_v, dst_v, et_v, qidx_v, q_v, k_v, b_v, ex_v,
                     rows_v, acc, sem1, sem2, sem3, sem4):
    c = lax.axis_index("c")
    s = lax.axis_index("s")
    wid = c * 16 + s
    _zero_rows128(rows_v)
    _init_acc(rows_v, acc, s)
    plsc.subcore_barrier()

    def chunk(ch, _):
        base = wid * EPW + ch * CH
        pltpu.sync_copy(src_hbm.at[pl.ds(base, CH)], src_v)
        pltpu.sync_copy(dst_hbm.at[pl.ds(base, CH)], dst_v)
        pltpu.sync_copy(et_hbm.at[pl.ds(base, CH)], et_v)
        cp4 = pltpu.async_copy(ones_hbm.at[et_v], rows_v, sem4)
        for j in range(CH // 16):
            sl = pl.ds(j * 16, 16)
            qidx_v[sl] = dst_v[sl] * 16 + et_v[sl]
        cp1 = pltpu.async_copy(qflat_hbm.at[qidx_v], q_v, sem1)
        cp3 = pltpu.async_copy(bflat_hbm.at[dst_v], b_v, sem3)
        cp1.wait()
        for j in range(CH // 16):
            sl = pl.ds(j * 16, 16)
            qidx_v[sl] = src_v[sl] * 16 + et_v[sl]
        cp2 = pltpu.async_copy(kflat_hbm.at[qidx_v], k_v, sem2)
        cp2.wait()
        cp3.wait()
        for j in range(CH // 16):
            sl = pl.ds(j * 16, 16)
            l = q_v[sl] + k_v[sl]
            l = jnp.where(l >= 0.0, l, 0.2 * l)
            ex_v[sl] = jnp.exp(l - b_v[sl])
        cp4.wait()
        for j in range(CH // 16):
            w16 = ex_v[pl.ds(j * 16, 16)]
            for t in range(16):
                ws = _splat(w16, t)
                for f in range(H // 16):
                    sl = pl.ds(f * 16, 16)
                    rows_v[j * 16 + t, sl] = rows_v[j * 16 + t, sl] * ws
        pltpu.sync_copy(rows_v, acc.at[dst_v], add=True)
        pltpu.sync_copy(ex_v, ex_hbm.at[pl.ds(base, CH)])
        return 0

    lax.fori_loop(0, NCHUNK, chunk, 0)
    plsc.subcore_barrier()
    _drain(acc, sp_hbm, c, s)

def _xall_body(x_ref, w_ref, o_ref):
    o_ref[0] = jnp.dot(x_ref[...], w_ref[0], preferred_element_type=jnp.float32)


def _xall(h, W):
    out = pl.pallas_call(
        _xall_body,
        grid=(R,),
        in_specs=[
            pl.BlockSpec((N, F), lambda r: (0, 0)),
            pl.BlockSpec((1, F, H), lambda r: (r, 0, 0)),
        ],
        out_specs=pl.BlockSpec((1, N, H), lambda r: (r, 0, 0)),
        out_shape=jax.ShapeDtypeStruct((R, N, H), jnp.float32),
    )(h, W)
    return out.reshape(R * N, H)


def _inv_body(p_ref, o_ref):
    cnt = p_ref[0, :, :16] + p_ref[1, :, :16]
    o_ref[...] = 1.0 / jnp.maximum(cnt, 1.0)


def _inv_table(parts):
    return pl.pallas_call(
        _inv_body,
        out_shape=jax.ShapeDtypeStruct((NPAD, 16), jnp.float32),
    )(parts)


def _epi_rgcn_body(p_ref, h_ref, root_ref, b_ref, g_ref, be_ref, o_ref):
    agg = p_ref[0, :N, :] + p_ref[1, :N, :]
    agg = agg + jnp.dot(h_ref[...], root_ref[...],
                        preferred_element_type=jnp.float32) + b_ref[...]
    mu = jnp.mean(agg, axis=0, keepdims=True)
    var = jnp.mean((agg - mu) ** 2, axis=0, keepdims=True)
    y = (agg - mu) / jnp.sqrt(var + 1e-5) * g_ref[...] + be_ref[...]
    o_ref[...] = jnp.maximum(y, 0.0)


def _epi_rgcn(parts, h, Root, b, g, be):
    return pl.pallas_call(
        _epi_rgcn_body,
        out_shape=jax.ShapeDtypeStruct((N, H), jnp.float32),
    )(parts, h, Root, b.reshape(1, H), g.reshape(1, H), be.reshape(1, H))


def _prep3_body(h_ref, w3_ref, aq_ref, ak_ref, q_ref, k_ref, b_ref):
    h = h_ref[...]
    vq = jnp.einsum("rhk,rk->rh", w3_ref[...], aq_ref[...],
                    preferred_element_type=jnp.float32)
    vk = jnp.einsum("rhk,rk->rh", w3_ref[...], ak_ref[...],
                    preferred_element_type=jnp.float32)
    pad = jnp.zeros((16 - R, H), jnp.float32)
    vq16 = jnp.concatenate([vq, pad], axis=0)
    vk16 = jnp.concatenate([vk, pad], axis=0)
    q = jnp.dot(h, vq16.T, preferred_element_type=jnp.float32)
    k = jnp.dot(h, vk16.T, preferred_element_type=jnp.float32)
    lane = lax.broadcasted_iota(jnp.int32, (1, 16), 1)
    valid = lane < R
    kmax = jnp.max(jnp.where(valid, k, -jnp.inf), axis=0, keepdims=True)
    bpre = jnp.max(jnp.where(valid, q + kmax, -jnp.inf), axis=1, keepdims=True)
    bnd = jnp.where(bpre >= 0.0, bpre, 0.2 * bpre)
    q_ref[...] = jnp.where(valid, q, 0.0)
    k_ref[...] = jnp.where(valid, k, 0.0)
    b_ref[...] = bnd


def _prep3(h, W3, aq, ak):
    return pl.pallas_call(
        _prep3_body,
        out_shape=(
            jax.ShapeDtypeStruct((N, 16), jnp.float32),
            jax.ShapeDtypeStruct((N, 16), jnp.float32),
            jax.ShapeDtypeStruct((N, 1), jnp.float32),
        ),
    )(h, W3, aq, ak)


def _sinv_body(p_ref, o_ref):
    stot = p_ref[0, :, :1] + p_ref[1, :, :1]
    o_ref[...] = 1.0 / jnp.maximum(stot, 1e-16)


def _sinv_table(parts):
    return pl.pallas_call(
        _sinv_body,
        out_shape=jax.ShapeDtypeStruct((NPAD, 1), jnp.float32),
    )(parts)


def _epi3_pool_body(p_ref, b_ref, g_ref, be_ref, batch_ref, wl_ref, bl_ref,
                    o_ref):
    agg = p_ref[0, :N, :] + p_ref[1, :N, :] + b_ref[...]
    mu = jnp.mean(agg, axis=0, keepdims=True)
    var = jnp.mean((agg - mu) ** 2, axis=0, keepdims=True)
    y = (agg - mu) / jnp.sqrt(var + 1e-5) * g_ref[...] + be_ref[...]
    h = jnp.maximum(y, 0.0)
    gid = lax.broadcasted_iota(jnp.int32, (G, N), 0)
    P = (batch_ref[...] == gid).astype(jnp.float32)
    cnt = jnp.sum(P, axis=1, keepdims=True)
    pooled = jnp.dot(P, h, preferred_element_type=jnp.float32)
    pooled = pooled / jnp.maximum(cnt, 1.0)
    o_ref[...] = jnp.dot(pooled, wl_ref[...],
                         preferred_element_type=jnp.float32) + bl_ref[...]


def _epi3_pool(parts, b3, g3, be3, batch, Wl, bl):
    return pl.pallas_call(
        _epi3_pool_body,
        out_shape=jax.ShapeDtypeStruct((G, C), jnp.float32),
    )(parts, b3.reshape(1, H), g3.reshape(1, H), be3.reshape(1, H),
      batch.reshape(1, N), Wl, bl.reshape(1, C))


def kernel(x, edge_index, edge_type, batch, W1, Root1, b1, g1, be1,
           W2, Root2, b2, g2, be2, W3, aq, ak, b3, g3, be3, Wl, bl):
    src = edge_index[0]
    dst = edge_index[1]
    et = edge_type
    eye128 = jnp.concatenate(
        [jnp.eye(16, dtype=jnp.float32),
         jnp.zeros((16, H - 16), jnp.float32)], axis=1)
    ones128 = jnp.ones((16, H), jnp.float32)

    cnt_parts = _sc_cnt(eye128, dst, et)
    wflat = _inv_table(cnt_parts).reshape(NPAD * 16)

    xall1 = _xall(x, W1)
    p1 = _sc_agg_rgcn(xall1, wflat, src, dst, et)
    h1 = _epi_rgcn(p1, x, Root1, b1, g1, be1)

    xall2 = _xall(h1, W2)
    p2 = _sc_agg_rgcn(xall2, wflat, src, dst, et)
    h2 = _epi_rgcn(p2, h1, Root2, b2, g2, be2)

    q16, k16, bnd = _prep3(h2, W3, aq, ak)
    qflat = q16.reshape(N * 16)
    kflat = k16.reshape(N * 16)
    bflat = bnd.reshape(N)
    ex, s_parts = _sc_logits_fused(ones128, qflat, kflat, bflat, src, dst, et)
    sinv = _sinv_table(s_parts).reshape(NPAD)
    xall3 = _xall(h2, W3)
    p3 = _sc_agg_rgat(xall3, sinv, src, dst, et, ex)
    return _epi3_pool(p3, b3, g3, be3, batch, Wl, bl)


# speedup vs baseline: 6.8979x; 1.0331x over previous
import functools

import jax
import jax.numpy as jnp
from jax import lax
from jax.experimental import pallas as pl
from jax.experimental.pallas import tpu as pltpu
from jax.experimental.pallas import tpu_sc as plsc

N = 10000
E = 320000
R = 8
F = 128
H = 128
C = 16
G = 64

NPAD = 10240
NW = 32
EPW = E // NW
CH = 80
NCHUNK = EPW // CH
RPS = NPAD // 16

_mesh = plsc.VectorSubcoreMesh(core_axis_name="c", subcore_axis_name="s")


def _splat(v16, t):
    return jnp.zeros((16,), v16.dtype) + v16[t]


def _zero_rows128(rows_v):
    z = jnp.zeros((16,), jnp.float32)
    for i in range(CH):
        for f in range(H // 16):
            rows_v[i, pl.ds(f * 16, 16)] = z


def _init_acc(rows_v, acc, s):
    for j in range(RPS // CH):
        pltpu.sync_copy(rows_v, acc.at[pl.ds(s * RPS + j * CH, CH)])


def _drain(acc, out_hbm, c, s):
    pltpu.sync_copy(acc.at[pl.ds(s * RPS, RPS)],
                    out_hbm.at[c].at[pl.ds(s * RPS, RPS)])


@functools.partial(
    pl.kernel,
    mesh=_mesh,
    out_type=jax.ShapeDtypeStruct((2, NPAD, H), jnp.float32),
    scratch_types=[
        pltpu.VMEM((CH,), jnp.int32), pltpu.VMEM((CH,), jnp.int32),
        pltpu.VMEM((CH,), jnp.int32), pltpu.VMEM((CH,), jnp.int32),
        pltpu.VMEM((CH, H), jnp.float32), pltpu.VMEM((CH, H), jnp.float32),
        pltpu.VMEM_SHARED((NPAD, H), jnp.float32),
        pltpu.SemaphoreType.DMA, pltpu.SemaphoreType.DMA,
    ],
)
def _sc_cnt(eye_hbm, dst_hbm, et_hbm, out_hbm,
            dst0, dst1, et0, et1, rows0, rows1, acc, sem0, sem1):
    c = lax.axis_index("c")
    s = lax.axis_index("s")
    wid = c * 16 + s
    _zero_rows128(rows0)
    _init_acc(rows0, acc, s)
    plsc.subcore_barrier()

    bank = ((dst0, et0, rows0, sem0), (dst1, et1, rows1, sem1))

    def load_edges(ch, b):
        dv, ev, rv, sm = bank[b]
        base = wid * EPW + ch * CH
        pltpu.sync_copy(dst_hbm.at[pl.ds(base, CH)], dv)
        pltpu.sync_copy(et_hbm.at[pl.ds(base, CH)], ev)
        pltpu.async_copy(eye_hbm.at[ev], rv, sm)

    def process(b):
        dv, ev, rv, sm = bank[b]
        pltpu.make_async_copy(eye_hbm.at[ev], rv, sm).wait()
        pltpu.sync_copy(rv, acc.at[dv], add=True)

    load_edges(0, 0)

    def chunk(g, _):
        load_edges(2 * g + 1, 1)
        process(0)
        load_edges(2 * g + 2, 0)
        process(1)
        return 0

    lax.fori_loop(0, (NCHUNK - 1) // 2, chunk, 0)
    process(0)
    plsc.subcore_barrier()
    _drain(acc, out_hbm, c, s)


@functools.partial(
    pl.kernel,
    mesh=_mesh,
    out_type=jax.ShapeDtypeStruct((2, NPAD, H), jnp.float32),
    scratch_types=[
        pltpu.VMEM((CH,), jnp.int32), pltpu.VMEM((CH,), jnp.int32),
        pltpu.VMEM((CH,), jnp.int32), pltpu.VMEM((CH,), jnp.int32),
        pltpu.VMEM((CH,), jnp.int32), pltpu.VMEM((CH,), jnp.int32),
        pltpu.VMEM((CH,), jnp.int32), pltpu.VMEM((CH,), jnp.int32),
        pltpu.VMEM((CH,), jnp.int32), pltpu.VMEM((CH,), jnp.int32),
        pltpu.VMEM((CH,), jnp.float32), pltpu.VMEM((CH,), jnp.float32),
        pltpu.VMEM((CH, H), jnp.float32), pltpu.VMEM((CH, H), jnp.float32),
        pltpu.VMEM_SHARED((NPAD, H), jnp.float32),
        pltpu.SemaphoreType.DMA, pltpu.SemaphoreType.DMA,
        pltpu.SemaphoreType.DMA, pltpu.SemaphoreType.DMA,
    ],
)
def _sc_agg_rgcn(xall_hbm, wflat_hbm, src_hbm, dst_hbm, et_hbm, out_hbm,
                 src0, src1, dst0, dst1, et0, et1, idx0, idx1, wix0, wix1,
                 w0, w1, rows0, rows1, acc,
                 semr0, semw0, semr1, semw1):
    c = lax.axis_index("c")
    s = lax.axis_index("s")
    wid = c * 16 + s
    _zero_rows128(rows0)
    _init_acc(rows0, acc, s)
    plsc.subcore_barrier()

    bank = ((src0, dst0, et0, idx0, wix0, w0, rows0, semr0, semw0),
            (src1, dst1, et1, idx1, wix1, w1, rows1, semr1, semw1))

    def load_edges(ch, b):
        sv, dv, ev, iv, wv, wlv, rv, sr, sw = bank[b]
        base = wid * EPW + ch * CH
        pltpu.sync_copy(src_hbm.at[pl.ds(base, CH)], sv)
        pltpu.sync_copy(dst_hbm.at[pl.ds(base, CH)], dv)
        pltpu.sync_copy(et_hbm.at[pl.ds(base, CH)], ev)
        for j in range(CH // 16):
            sl = pl.ds(j * 16, 16)
            iv[sl] = ev[sl] * N + sv[sl]
            wv[sl] = dv[sl] * 16 + ev[sl]
        pltpu.async_copy(xall_hbm.at[iv], rv, sr)
        pltpu.async_copy(wflat_hbm.at[wv], wlv, sw)

    def process(b):
        sv, dv, ev, iv, wv, wlv, rv, sr, sw = bank[b]
        pltpu.make_async_copy(wflat_hbm.at[wv], wlv, sw).wait()
        pltpu.make_async_copy(xall_hbm.at[iv], rv, sr).wait()
        for j in range(CH // 16):
            w16 = wlv[pl.ds(j * 16, 16)]
            for t in range(16):
                ws = _splat(w16, t)
                for f in range(H // 16):
                    sl = pl.ds(f * 16, 16)
                    rv[j * 16 + t, sl] = rv[j * 16 + t, sl] * ws
        pltpu.sync_copy(rv, acc.at[dv], add=True)

    load_edges(0, 0)

    def chunk(g, _):
        load_edges(2 * g + 1, 1)
        process(0)
        load_edges(2 * g + 2, 0)
        process(1)
        return 0

    lax.fori_loop(0, (NCHUNK - 1) // 2, chunk, 0)
    process(0)
    plsc.subcore_barrier()
    _drain(acc, out_hbm, c, s)


@functools.partial(
    pl.kernel,
    mesh=_mesh,
    out_type=jax.ShapeDtypeStruct((E,), jnp.float32),
    scratch_types=[
        pltpu.VMEM((CH,), jnp.int32),
        pltpu.VMEM((CH,), jnp.int32),
        pltpu.VMEM((CH,), jnp.int32),
        pltpu.VMEM((CH,), jnp.int32),
        pltpu.VMEM((CH,), jnp.float32),
        pltpu.VMEM((CH,), jnp.float32),
        pltpu.VMEM((CH,), jnp.float32),
        pltpu.VMEM((CH,), jnp.float32),
        pltpu.SemaphoreType.DMA,
        pltpu.SemaphoreType.DMA,
        pltpu.SemaphoreType.DMA,
    ],
)
def _sc_logits_ex(qflat_hbm, kflat_hbm, bflat_hbm, src_hbm, dst_hbm, et_hbm,
                  ex_hbm,
                  src_v, dst_v, et_v, qidx_v, q_v, k_v, b_v, ex_v,
                  sem1, sem2, sem3):
    c = lax.axis_index("c")
    s = lax.axis_index("s")
    wid = c * 16 + s

    def chunk(ch, _):
        base = wid * EPW + ch * CH
        pltpu.sync_copy(src_hbm.at[pl.ds(base, CH)], src_v)
        pltpu.sync_copy(dst_hbm.at[pl.ds(base, CH)], dst_v)
        pltpu.sync_copy(et_hbm.at[pl.ds(base, CH)], et_v)
        for j in range(CH // 16):
            sl = pl.ds(j * 16, 16)
            qidx_v[sl] = dst_v[sl] * 16 + et_v[sl]
        cp1 = pltpu.async_copy(qflat_hbm.at[qidx_v], q_v, sem1)
        cp3 = pltpu.async_copy(bflat_hbm.at[dst_v], b_v, sem3)
        cp1.wait()
        for j in range(CH // 16):
            sl = pl.ds(j * 16, 16)
            qidx_v[sl] = src_v[sl] * 16 + et_v[sl]
        cp2 = pltpu.async_copy(kflat_hbm.at[qidx_v], k_v, sem2)
        cp2.wait()
        cp3.wait()
        for j in range(CH // 16):
            sl = pl.ds(j * 16, 16)
            l = q_v[sl] + k_v[sl]
            l = jnp.where(l >= 0.0, l, 0.2 * l)
            ex_v[sl] = jnp.exp(l - b_v[sl])
        pltpu.sync_copy(ex_v, ex_hbm.at[pl.ds(base, CH)])
        return 0

    lax.fori_loop(0, NCHUNK, chunk, 0)


@functools.partial(
    pl.kernel,
    mesh=_mesh,
    out_type=jax.ShapeDtypeStruct((2, NPAD, H), jnp.float32),
    scratch_types=[
        pltpu.VMEM((CH,), jnp.int32),
        pltpu.VMEM((CH,), jnp.int32),
        pltpu.VMEM((CH,), jnp.float32),
        pltpu.VMEM((CH, H), jnp.float32),
        pltpu.VMEM_SHARED((NPAD, H), jnp.float32),
        pltpu.SemaphoreType.DMA,
    ],
)
def _sc_exsum(ones_hbm, dst_hbm, et_hbm, ex_hbm, out_hbm,
              dst_v, et_v, ex_v, rows_v, acc, sem):
    c = lax.axis_index("c")
    s = lax.axis_index("s")
    wid = c * 16 + s
    _zero_rows128(rows_v)
    _init_acc(rows_v, acc, s)
    plsc.subcore_barrier()

    def chunk(ch, _):
        base = wid * EPW + ch * CH
        pltpu.sync_copy(dst_hbm.at[pl.ds(base, CH)], dst_v)
        pltpu.sync_copy(et_hbm.at[pl.ds(base, CH)], et_v)
        pltpu.sync_copy(ex_hbm.at[pl.ds(base, CH)], ex_v)
        pltpu.async_copy(ones_hbm.at[et_v], rows_v, sem).wait()
        for j in range(CH // 16):
            w16 = ex_v[pl.ds(j * 16, 16)]
            for t in range(16):
                ws = _splat(w16, t)
                for f in range(H // 16):
                    sl = pl.ds(f * 16, 16)
                    rows_v[j * 16 + t, sl] = rows_v[j * 16 + t, sl] * ws
        pltpu.sync_copy(rows_v, acc.at[dst_v], add=True)
        return 0

    lax.fori_loop(0, NCHUNK, chunk, 0)
    plsc.subcore_barrier()
    _drain(acc, out_hbm, c, s)


@functools.partial(
    pl.kernel,
    mesh=_mesh,
    out_type=jax.ShapeDtypeStruct((2, NPAD, H), jnp.float32),
    scratch_types=[
        pltpu.VMEM((CH,), jnp.int32), pltpu.VMEM((CH,), jnp.int32),
        pltpu.VMEM((CH,), jnp.int32), pltpu.VMEM((CH,), jnp.int32),
        pltpu.VMEM((CH,), jnp.int32), pltpu.VMEM((CH,), jnp.int32),
        pltpu.VMEM((CH,), jnp.int32), pltpu.VMEM((CH,), jnp.int32),
        pltpu.VMEM((CH,), jnp.float32), pltpu.VMEM((CH,), jnp.float32),
        pltpu.VMEM((CH,), jnp.float32), pltpu.VMEM((CH,), jnp.float32),
        pltpu.VMEM((CH, H), jnp.float32), pltpu.VMEM((CH, H), jnp.float32),
        pltpu.VMEM_SHARED((NPAD, H), jnp.float32),
        pltpu.SemaphoreType.DMA, pltpu.SemaphoreType.DMA,
        pltpu.SemaphoreType.DMA, pltpu.SemaphoreType.DMA,
    ],
)
def _sc_agg_rgat(xall_hbm, sinv_hbm, src_hbm, dst_hbm, et_hbm, ex_hbm,
                 out_hbm,
                 src0, src1, dst0, dst1, et0, et1, idx0, idx1,
                 ex0, ex1, sv0, sv1, rows0, rows1, acc,
                 semr0, semw0, semr1, semw1):
    c = lax.axis_index("c")
    s = lax.axis_index("s")
    wid = c * 16 + s
    _zero_rows128(rows0)
    _init_acc(rows0, acc, s)
    plsc.subcore_barrier()

    bank = ((src0, dst0, et0, idx0, ex0, sv0, rows0, semr0, semw0),
            (src1, dst1, et1, idx1, ex1, sv1, rows1, semr1, semw1))

    def load_edges(ch, b):
        sv_, dv, ev, iv, exv, svv, rv, sr, sw = bank[b]
        base = wid * EPW + ch * CH
        pltpu.sync_copy(src_hbm.at[pl.ds(base, CH)], sv_)
        pltpu.sync_copy(dst_hbm.at[pl.ds(base, CH)], dv)
        pltpu.sync_copy(et_hbm.at[pl.ds(base, CH)], ev)
        pltpu.sync_copy(ex_hbm.at[pl.ds(base, CH)], exv)
        for j in range(CH // 16):
            sl = pl.ds(j * 16, 16)
            iv[sl] = ev[sl] * N + sv_[sl]
        pltpu.async_copy(xall_hbm.at[iv], rv, sr)
        pltpu.async_copy(sinv_hbm.at[dv], svv, sw)

    def process(b):
        sv_, dv, ev, iv, exv, svv, rv, sr, sw = bank[b]
        pltpu.make_async_copy(sinv_hbm.at[dv], svv, sw).wait()
        pltpu.make_async_copy(xall_hbm.at[iv], rv, sr).wait()
        for j in range(CH // 16):
            sl = pl.ds(j * 16, 16)
            exv[sl] = exv[sl] * svv[sl]
        for j in range(CH // 16):
            w16 = exv[pl.ds(j * 16, 16)]
            for t in range(16):
                ws = _splat(w16, t)
                for f in range(H // 16):
                    sl = pl.ds(f * 16, 16)
                    rv[j * 16 + t, sl] = rv[j * 16 + t, sl] * ws
        pltpu.sync_copy(rv, acc.at[dv], add=True)

    load_edges(0, 0)

    def chunk(g, _):
        load_edges(2 * g + 1, 1)
        process(0)
        load_edges(2 * g + 2, 0)
        process(1)
        return 0

    lax.fori_loop(0, (NCHUNK - 1) // 2, chunk, 0)
    process(0)
    plsc.subcore_barrier()
    _drain(acc, out_hbm, c, s)


@functools.partial(
    pl.kernel,
    mesh=_mesh,
    out_type=(
        jax.ShapeDtypeStruct((E,), jnp.float32),
        jax.ShapeDtypeStruct((2, NPAD, H), jnp.float32),
    ),
    scratch_types=[
        pltpu.VMEM((CH,), jnp.int32), pltpu.VMEM((CH,), jnp.int32),
        pltpu.VMEM((CH,), jnp.int32), pltpu.VMEM((CH,), jnp.int32),
        pltpu.VMEM((CH,), jnp.int32), pltpu.VMEM((CH,), jnp.int32),
        pltpu.VMEM((CH,), jnp.int32), pltpu.VMEM((CH,), jnp.int32),
        pltpu.VMEM((CH,), jnp.int32), pltpu.VMEM((CH,), jnp.int32),
        pltpu.VMEM((CH,), jnp.float32), pltpu.VMEM((CH,), jnp.float32),
        pltpu.VMEM((CH,), jnp.float32), pltpu.VMEM((CH,), jnp.float32),
        pltpu.VMEM((CH,), jnp.float32), pltpu.VMEM((CH,), jnp.float32),
        pltpu.VMEM((CH,), jnp.float32), pltpu.VMEM((CH,), jnp.float32),
        pltpu.VMEM((CH, H), jnp.float32), pltpu.VMEM((CH, H), jnp.float32),
        pltpu.VMEM_SHARED((NPAD, H), jnp.float32),
        pltpu.SemaphoreType.DMA, pltpu.SemaphoreType.DMA,
        pltpu.SemaphoreType.DMA, pltpu.SemaphoreType.DMA,
        pltpu.SemaphoreType.DMA, pltpu.SemaphoreType.DMA,
        pltpu.SemaphoreType.DMA, pltpu.SemaphoreType.DMA,
    ],
)
def _sc_logits_fused(ones_hbm, qflat_hbm, kflat_hbm, bflat_hbm, src_hbm,
                     dst_hbm, et_hbm, ex_hbm, sp_hbm,
                     src0, src1, dst0, dst1, et0, et1, qix0, qix1, kix0, kix1,
                     q0, q1, k0, k1, b0, b1, ex0, ex1, rows0, rows1, acc,
                     smq0, smq1, smk0, smk1, smb0, smb1, smr0, smr1):
    c = lax.axis_index("c")
    s = lax.axis_index("s")
    wid = c * 16 + s
    _zero_rows128(rows0)
    _init_acc(rows0, acc, s)
    plsc.subcore_barrier()

    bank = ((src0, dst0, et0, qix0, kix0, q0, k0, b0, ex0, rows0,
             smq0, smk0, smb0, smr0),
            (src1, dst1, et1, qix1, kix1, q1, k1, b1, ex1, rows1,
             smq1, smk1, smb1, smr1))

    def load_edges(ch, b):
        sv, dv, ev, qi, ki, qv, kv, bv, exv, rv, smq, smk, smb, smr = bank[b]
        base = wid * EPW + ch * CH
        pltpu.sync_copy(src_hbm.at[pl.ds(base, CH)], sv)
        pltpu.sync_copy(dst_hbm.at[pl.ds(base, CH)], dv)
        pltpu.sync_copy(et_hbm.at[pl.ds(base, CH)], ev)
        for j in range(CH // 16):
            sl = pl.ds(j * 16, 16)
            qi[sl] = dv[sl] * 16 + ev[sl]
            ki[sl] = sv[sl] * 16 + ev[sl]
        pltpu.async_copy(qflat_hbm.at[qi], qv, smq)
        pltpu.async_copy(kflat_hbm.at[ki], kv, smk)
        pltpu.async_copy(bflat_hbm.at[dv], bv, smb)
        pltpu.async_copy(ones_hbm.at[ev], rv, smr)

    def process(ch, b):
        sv, dv, ev, qi, ki, qv, kv, bv, exv, rv, smq, smk, smb, smr = bank[b]
        base = wid * EPW + ch * CH
        pltpu.make_async_copy(qflat_hbm.at[qi], qv, smq).wait()
        pltpu.make_async_copy(kflat_hbm.at[ki], kv, smk).wait()
        pltpu.make_async_copy(bflat_hbm.at[dv], bv, smb).wait()
        pltpu.make_async_copy(ones_hbm.at[ev], rv, smr).wait()
        for j in range(CH // 16):
            sl = pl.ds(j * 16, 16)
            l = qv[sl] + kv[sl]
            l = jnp.where(l >= 0.0, l, 0.2 * l)
            exv[sl] = jnp.exp(l - bv[sl])
        for j in range(CH // 16):
            w16 = exv[pl.ds(j * 16, 16)]
            for t in range(16):
                ws = _splat(w16, t)
                for f in range(H // 16):
                    sl = pl.ds(f * 16, 16)
                    rv[j * 16 + t, sl] = rv[j * 16 + t, sl] * ws
        pltpu.sync_copy(rv, acc.at[dv], add=True)
        pltpu.sync_copy(exv, ex_hbm.at[pl.ds(base, CH)])

    load_edges(0, 0)

    def chunk(g, _):
        load_edges(2 * g + 1, 1)
        process(2 * g, 0)
        load_edges(2 * g + 2, 0)
        process(2 * g + 1, 1)
        return 0

    lax.fori_loop(0, (NCHUNK - 1) // 2, chunk, 0)
    process(NCHUNK - 1, 0)
    plsc.subcore_barrier()
    _drain(acc, sp_hbm, c, s)


def _xall_body(x_ref, w_ref, o_ref):
    o_ref[0] = jnp.dot(x_ref[...], w_ref[0], preferred_element_type=jnp.float32)


def _xall(h, W):
    out = pl.pallas_call(
        _xall_body,
        grid=(R,),
        in_specs=[
            pl.BlockSpec((N, F), lambda r: (0, 0)),
            pl.BlockSpec((1, F, H), lambda r: (r, 0, 0)),
        ],
        out_specs=pl.BlockSpec((1, N, H), lambda r: (r, 0, 0)),
        out_shape=jax.ShapeDtypeStruct((R, N, H), jnp.float32),
    )(h, W)
    return out.reshape(R * N, H)


def _inv_body(p_ref, o_ref):
    cnt = p_ref[0, :, :16] + p_ref[1, :, :16]
    o_ref[...] = 1.0 / jnp.maximum(cnt, 1.0)


def _inv_table(parts):
    return pl.pallas_call(
        _inv_body,
        out_shape=jax.ShapeDtypeStruct((NPAD, 16), jnp.float32),
    )(parts)


def _epi_rgcn_body(p_ref, h_ref, root_ref, b_ref, g_ref, be_ref, o_ref):
    agg = p_ref[0, :N, :] + p_ref[1, :N, :]
    agg = agg + jnp.dot(h_ref[...], root_ref[...],
                        preferred_element_type=jnp.float32) + b_ref[...]
    mu = jnp.mean(agg, axis=0, keepdims=True)
    var = jnp.mean((agg - mu) ** 2, axis=0, keepdims=True)
    y = (agg - mu) / jnp.sqrt(var + 1e-5) * g_ref[...] + be_ref[...]
    o_ref[...] = jnp.maximum(y, 0.0)


def _epi_rgcn(parts, h, Root, b, g, be):
    return pl.pallas_call(
        _epi_rgcn_body,
        out_shape=jax.ShapeDtypeStruct((N, H), jnp.float32),
    )(parts, h, Root, b.reshape(1, H), g.reshape(1, H), be.reshape(1, H))


def _prep3_body(h_ref, w3_ref, aq_ref, ak_ref, q_ref, k_ref, b_ref):
    h = h_ref[...]
    vq = jnp.einsum("rhk,rk->rh", w3_ref[...], aq_ref[...],
                    preferred_element_type=jnp.float32)
    vk = jnp.einsum("rhk,rk->rh", w3_ref[...], ak_ref[...],
                    preferred_element_type=jnp.float32)
    pad = jnp.zeros((16 - R, H), jnp.float32)
    vq16 = jnp.concatenate([vq, pad], axis=0)
    vk16 = jnp.concatenate([vk, pad], axis=0)
    q = jnp.dot(h, vq16.T, preferred_element_type=jnp.float32)
    k = jnp.dot(h, vk16.T, preferred_element_type=jnp.float32)
    lane = lax.broadcasted_iota(jnp.int32, (1, 16), 1)
    valid = lane < R
    kmax = jnp.max(jnp.where(valid, k, -jnp.inf), axis=0, keepdims=True)
    bpre = jnp.max(jnp.where(valid, q + kmax, -jnp.inf), axis=1, keepdims=True)
    bnd = jnp.where(bpre >= 0.0, bpre, 0.2 * bpre)
    q_ref[...] = jnp.where(valid, q, 0.0)
    k_ref[...] = jnp.where(valid, k, 0.0)
    b_ref[...] = bnd


def _prep3(h, W3, aq, ak):
    return pl.pallas_call(
        _prep3_body,
        out_shape=(
            jax.ShapeDtypeStruct((N, 16), jnp.float32),
            jax.ShapeDtypeStruct((N, 16), jnp.float32),
            jax.ShapeDtypeStruct((N, 1), jnp.float32),
        ),
    )(h, W3, aq, ak)


def _sinv_body(p_ref, o_ref):
    stot = p_ref[0, :, :1] + p_ref[1, :, :1]
    o_ref[...] = 1.0 / jnp.maximum(stot, 1e-16)


def _sinv_table(parts):
    return pl.pallas_call(
        _sinv_body,
        out_shape=jax.ShapeDtypeStruct((NPAD, 1), jnp.float32),
    )(parts)


def _epi3_pool_body(p_ref, b_ref, g_ref, be_ref, batch_ref, wl_ref, bl_ref,
                    o_ref):
    agg = p_ref[0, :N, :] + p_ref[1, :N, :] + b_ref[...]
    mu = jnp.mean(agg, axis=0, keepdims=True)
    var = jnp.mean((agg - mu) ** 2, axis=0, keepdims=True)
    y = (agg - mu) / jnp.sqrt(var + 1e-5) * g_ref[...] + be_ref[...]
    h = jnp.maximum(y, 0.0)
    gid = lax.broadcasted_iota(jnp.int32, (G, N), 0)
    P = (batch_ref[...] == gid).astype(jnp.float32)
    cnt = jnp.sum(P, axis=1, keepdims=True)
    pooled = jnp.dot(P, h, preferred_element_type=jnp.float32)
    pooled = pooled / jnp.maximum(cnt, 1.0)
    o_ref[...] = jnp.dot(pooled, wl_ref[...],
                         preferred_element_type=jnp.float32) + bl_ref[...]


def _epi3_pool(parts, b3, g3, be3, batch, Wl, bl):
    return pl.pallas_call(
        _epi3_pool_body,
        out_shape=jax.ShapeDtypeStruct((G, C), jnp.float32),
    )(parts, b3.reshape(1, H), g3.reshape(1, H), be3.reshape(1, H),
      batch.reshape(1, N), Wl, bl.reshape(1, C))


def kernel(x, edge_index, edge_type, batch, W1, Root1, b1, g1, be1,
           W2, Root2, b2, g2, be2, W3, aq, ak, b3, g3, be3, Wl, bl):
    src = edge_index[0]
    dst = edge_index[1]
    et = edge_type
    eye128 = jnp.concatenate(
        [jnp.eye(16, dtype=jnp.float32),
         jnp.zeros((16, H - 16), jnp.float32)], axis=1)
    ones128 = jnp.ones((16, H), jnp.float32)

    cnt_parts = _sc_cnt(eye128, dst, et)
    wflat = _inv_table(cnt_parts).reshape(NPAD * 16)

    xall1 = _xall(x, W1)
    p1 = _sc_agg_rgcn(xall1, wflat, src, dst, et)
    h1 = _epi_rgcn(p1, x, Root1, b1, g1, be1)

    xall2 = _xall(h1, W2)
    p2 = _sc_agg_rgcn(xall2, wflat, src, dst, et)
    h2 = _epi_rgcn(p2, h1, Root2, b2, g2, be2)

    q16, k16, bnd = _prep3(h2, W3, aq, ak)
    qflat = q16.reshape(N * 16)
    kflat = k16.reshape(N * 16)
    bflat = bnd.reshape(N)
    ex, s_parts = _sc_logits_fused(ones128, qflat, kflat, bflat, src, dst, et)
    sinv = _sinv_table(s_parts).reshape(NPAD)
    xall3 = _xall(h2, W3)
    p3 = _sc_agg_rgat(xall3, sinv, src, dst, et, ex)
    return _epi3_pool(p3, b3, g3, be3, batch, Wl, bl)


# speedup vs baseline: 17.6205x; 2.5545x over previous
import functools

import jax
import jax.numpy as jnp
from jax import lax
from jax.experimental import pallas as pl
from jax.experimental.pallas import tpu as pltpu
from jax.experimental.pallas import tpu_sc as plsc

N = 10000
E = 320000
R = 8
F = 128
H = 128
C = 16
G = 64

NPAD = 10240
NW = 32
EPW = E // NW
CH = 80
NCHUNK = EPW // CH
RPS = NPAD // 16

_mesh = plsc.VectorSubcoreMesh(core_axis_name="c", subcore_axis_name="s")


def _splat(v16, t):
    return jnp.zeros((16,), v16.dtype) + v16[t]


def _zero_rows128(rows_v):
    z = jnp.zeros((16,), jnp.float32)
    for i in range(CH):
        for f in range(H // 16):
            rows_v[i, pl.ds(f * 16, 16)] = z


def _init_acc(rows_v, acc, s):
    for j in range(RPS // CH):
        pltpu.sync_copy(rows_v, acc.at[pl.ds(s * RPS + j * CH, CH)])


def _drain(acc, out_hbm, c, s):
    pltpu.sync_copy(acc.at[pl.ds(s * RPS, RPS)],
                    out_hbm.at[c].at[pl.ds(s * RPS, RPS)])


@functools.partial(
    pl.kernel,
    mesh=_mesh,
    out_type=jax.ShapeDtypeStruct((2, NPAD, H), jnp.float32),
    scratch_types=[
        pltpu.VMEM((CH,), jnp.int32), pltpu.VMEM((CH,), jnp.int32),
        pltpu.VMEM((CH,), jnp.int32), pltpu.VMEM((CH,), jnp.int32),
        pltpu.VMEM((CH, H), jnp.float32), pltpu.VMEM((CH, H), jnp.float32),
        pltpu.VMEM_SHARED((NPAD, H), jnp.float32),
        pltpu.VMEM_SHARED((16, H), jnp.float32),
        pltpu.SemaphoreType.DMA, pltpu.SemaphoreType.DMA,
    ],
)
def _sc_cnt(eye_hbm, dst_hbm, et_hbm, out_hbm,
            dst0, dst1, et0, et1, rows0, rows1, acc, eye_spm, sem0, sem1):
    c = lax.axis_index("c")
    s = lax.axis_index("s")
    wid = c * 16 + s
    _zero_rows128(rows0)
    _init_acc(rows0, acc, s)

    @pl.when(s == 0)
    def _():
        pltpu.sync_copy(eye_hbm, eye_spm)

    plsc.subcore_barrier()

    bank = ((dst0, et0, rows0, sem0), (dst1, et1, rows1, sem1))

    def load_edges(ch, b):
        dv, ev, rv, sm = bank[b]
        base = wid * EPW + ch * CH
        pltpu.sync_copy(dst_hbm.at[pl.ds(base, CH)], dv)
        pltpu.sync_copy(et_hbm.at[pl.ds(base, CH)], ev)
        pltpu.async_copy(eye_spm.at[ev], rv, sm)

    def process(b):
        dv, ev, rv, sm = bank[b]
        pltpu.make_async_copy(eye_spm.at[ev], rv, sm).wait()
        pltpu.sync_copy(rv, acc.at[dv], add=True)

    load_edges(0, 0)

    def chunk(g, _):
        load_edges(2 * g + 1, 1)
        process(0)
        load_edges(2 * g + 2, 0)
        process(1)
        return 0

    lax.fori_loop(0, (NCHUNK - 1) // 2, chunk, 0)
    process(0)
    plsc.subcore_barrier()
    _drain(acc, out_hbm, c, s)


@functools.partial(
    pl.kernel,
    mesh=_mesh,
    out_type=jax.ShapeDtypeStruct((2, NPAD, H), jnp.float32),
    scratch_types=[
        pltpu.VMEM((CH,), jnp.int32), pltpu.VMEM((CH,), jnp.int32),
        pltpu.VMEM((CH,), jnp.int32), pltpu.VMEM((CH,), jnp.int32),
        pltpu.VMEM((CH,), jnp.int32), pltpu.VMEM((CH,), jnp.int32),
        pltpu.VMEM((CH,), jnp.int32), pltpu.VMEM((CH,), jnp.int32),
        pltpu.VMEM((CH,), jnp.int32), pltpu.VMEM((CH,), jnp.int32),
        pltpu.VMEM((CH,), jnp.float32), pltpu.VMEM((CH,), jnp.float32),
        pltpu.VMEM((CH, H), jnp.float32), pltpu.VMEM((CH, H), jnp.float32),
        pltpu.VMEM_SHARED((NPAD, H), jnp.float32),
        pltpu.SemaphoreType.DMA, pltpu.SemaphoreType.DMA,
        pltpu.SemaphoreType.DMA, pltpu.SemaphoreType.DMA,
    ],
)
def _sc_agg_rgcn(xall_hbm, wflat_hbm, src_hbm, dst_hbm, et_hbm, out_hbm,
                 src0, src1, dst0, dst1, et0, et1, idx0, idx1, wix0, wix1,
                 w0, w1, rows0, rows1, acc,
                 semr0, semw0, semr1, semw1):
    c = lax.axis_index("c")
    s = lax.axis_index("s")
    wid = c * 16 + s
    _zero_rows128(rows0)
    _init_acc(rows0, acc, s)
    plsc.subcore_barrier()

    bank = ((src0, dst0, et0, idx0, wix0, w0, rows0, semr0, semw0),
            (src1, dst1, et1, idx1, wix1, w1, rows1, semr1, semw1))

    def load_edges(ch, b):
        sv, dv, ev, iv, wv, wlv, rv, sr, sw = bank[b]
        base = wid * EPW + ch * CH
        pltpu.sync_copy(src_hbm.at[pl.ds(base, CH)], sv)
        pltpu.sync_copy(dst_hbm.at[pl.ds(base, CH)], dv)
        pltpu.sync_copy(et_hbm.at[pl.ds(base, CH)], ev)
        for j in range(CH // 16):
            sl = pl.ds(j * 16, 16)
            iv[sl] = ev[sl] * N + sv[sl]
            wv[sl] = dv[sl] * 16 + ev[sl]
        pltpu.async_copy(xall_hbm.at[iv], rv, sr)
        pltpu.async_copy(wflat_hbm.at[wv], wlv, sw)

    def process(b):
        sv, dv, ev, iv, wv, wlv, rv, sr, sw = bank[b]
        pltpu.make_async_copy(wflat_hbm.at[wv], wlv, sw).wait()
        pltpu.make_async_copy(xall_hbm.at[iv], rv, sr).wait()
        for j in range(CH // 16):
            w16 = wlv[pl.ds(j * 16, 16)]
            for t in range(16):
                ws = _splat(w16, t)
                for f in range(H // 16):
                    sl = pl.ds(f * 16, 16)
                    rv[j * 16 + t, sl] = rv[j * 16 + t, sl] * ws
        pltpu.sync_copy(rv, acc.at[dv], add=True)

    load_edges(0, 0)

    def chunk(g, _):
        load_edges(2 * g + 1, 1)
        process(0)
        load_edges(2 * g + 2, 0)
        process(1)
        return 0

    lax.fori_loop(0, (NCHUNK - 1) // 2, chunk, 0)
    process(0)
    plsc.subcore_barrier()
    _drain(acc, out_hbm, c, s)


@functools.partial(
    pl.kernel,
    mesh=_mesh,
    out_type=jax.ShapeDtypeStruct((E,), jnp.float32),
    scratch_types=[
        pltpu.VMEM((CH,), jnp.int32),
        pltpu.VMEM((CH,), jnp.int32),
        pltpu.VMEM((CH,), jnp.int32),
        pltpu.VMEM((CH,), jnp.int32),
        pltpu.VMEM((CH,), jnp.float32),
        pltpu.VMEM((CH,), jnp.float32),
        pltpu.VMEM((CH,), jnp.float32),
        pltpu.VMEM((CH,), jnp.float32),
        pltpu.SemaphoreType.DMA,
        pltpu.SemaphoreType.DMA,
        pltpu.SemaphoreType.DMA,
    ],
)
def _sc_logits_ex(qflat_hbm, kflat_hbm, bflat_hbm, src_hbm, dst_hbm, et_hbm,
                  ex_hbm,
                  src_v, dst_v, et_v, qidx_v, q_v, k_v, b_v, ex_v,
                  sem1, sem2, sem3):
    c = lax.axis_index("c")
    s = lax.axis_index("s")
    wid = c * 16 + s

    def chunk(ch, _):
        base = wid * EPW + ch * CH
        pltpu.sync_copy(src_hbm.at[pl.ds(base, CH)], src_v)
        pltpu.sync_copy(dst_hbm.at[pl.ds(base, CH)], dst_v)
        pltpu.sync_copy(et_hbm.at[pl.ds(base, CH)], et_v)
        for j in range(CH // 16):
            sl = pl.ds(j * 16, 16)
            qidx_v[sl] = dst_v[sl] * 16 + et_v[sl]
        cp1 = pltpu.async_copy(qflat_hbm.at[qidx_v], q_v, sem1)
        cp3 = pltpu.async_copy(bflat_hbm.at[dst_v], b_v, sem3)
        cp1.wait()
        for j in range(CH // 16):
            sl = pl.ds(j * 16, 16)
            qidx_v[sl] = src_v[sl] * 16 + et_v[sl]
        cp2 = pltpu.async_copy(kflat_hbm.at[qidx_v], k_v, sem2)
        cp2.wait()
        cp3.wait()
        for j in range(CH // 16):
            sl = pl.ds(j * 16, 16)
            l = q_v[sl] + k_v[sl]
            l = jnp.where(l >= 0.0, l, 0.2 * l)
            ex_v[sl] = jnp.exp(l - b_v[sl])
        pltpu.sync_copy(ex_v, ex_hbm.at[pl.ds(base, CH)])
        return 0

    lax.fori_loop(0, NCHUNK, chunk, 0)


@functools.partial(
    pl.kernel,
    mesh=_mesh,
    out_type=jax.ShapeDtypeStruct((2, NPAD, H), jnp.float32),
    scratch_types=[
        pltpu.VMEM((CH,), jnp.int32),
        pltpu.VMEM((CH,), jnp.int32),
        pltpu.VMEM((CH,), jnp.float32),
        pltpu.VMEM((CH, H), jnp.float32),
        pltpu.VMEM_SHARED((NPAD, H), jnp.float32),
        pltpu.SemaphoreType.DMA,
    ],
)
def _sc_exsum(ones_hbm, dst_hbm, et_hbm, ex_hbm, out_hbm,
              dst_v, et_v, ex_v, rows_v, acc, sem):
    c = lax.axis_index("c")
    s = lax.axis_index("s")
    wid = c * 16 + s
    _zero_rows128(rows_v)
    _init_acc(rows_v, acc, s)
    plsc.subcore_barrier()

    def chunk(ch, _):
        base = wid * EPW + ch * CH
        pltpu.sync_copy(dst_hbm.at[pl.ds(base, CH)], dst_v)
        pltpu.sync_copy(et_hbm.at[pl.ds(base, CH)], et_v)
        pltpu.sync_copy(ex_hbm.at[pl.ds(base, CH)], ex_v)
        pltpu.async_copy(ones_hbm.at[et_v], rows_v, sem).wait()
        for j in range(CH // 16):
            w16 = ex_v[pl.ds(j * 16, 16)]
            for t in range(16):
                ws = _splat(w16, t)
                for f in range(H // 16):
                    sl = pl.ds(f * 16, 16)
                    rows_v[j * 16 + t, sl] = rows_v[j * 16 + t, sl] * ws
        pltpu.sync_copy(rows_v, acc.at[dst_v], add=True)
        return 0

    lax.fori_loop(0, NCHUNK, chunk, 0)
    plsc.subcore_barrier()
    _drain(acc, out_hbm, c, s)


@functools.partial(
    pl.kernel,
    mesh=_mesh,
    out_type=jax.ShapeDtypeStruct((2, NPAD, H), jnp.float32),
    scratch_types=[
        pltpu.VMEM((CH,), jnp.int32), pltpu.VMEM((CH,), jnp.int32),
        pltpu.VMEM((CH,), jnp.int32), pltpu.VMEM((CH,), jnp.int32),
        pltpu.VMEM((CH,), jnp.int32), pltpu.VMEM((CH,), jnp.int32),
        pltpu.VMEM((CH,), jnp.int32), pltpu.VMEM((CH,), jnp.int32),
        pltpu.VMEM((CH,), jnp.float32), pltpu.VMEM((CH,), jnp.float32),
        pltpu.VMEM((CH,), jnp.float32), pltpu.VMEM((CH,), jnp.float32),
        pltpu.VMEM((CH, H), jnp.float32), pltpu.VMEM((CH, H), jnp.float32),
        pltpu.VMEM_SHARED((NPAD, H), jnp.float32),
        pltpu.SemaphoreType.DMA, pltpu.SemaphoreType.DMA,
        pltpu.SemaphoreType.DMA, pltpu.SemaphoreType.DMA,
    ],
)
def _sc_agg_rgat(xall_hbm, sinv_hbm, src_hbm, dst_hbm, et_hbm, ex_hbm,
                 out_hbm,
                 src0, src1, dst0, dst1, et0, et1, idx0, idx1,
                 ex0, ex1, sv0, sv1, rows0, rows1, acc,
                 semr0, semw0, semr1, semw1):
    c = lax.axis_index("c")
    s = lax.axis_index("s")
    wid = c * 16 + s
    _zero_rows128(rows0)
    _init_acc(rows0, acc, s)
    plsc.subcore_barrier()

    bank = ((src0, dst0, et0, idx0, ex0, sv0, rows0, semr0, semw0),
            (src1, dst1, et1, idx1, ex1, sv1, rows1, semr1, semw1))

    def load_edges(ch, b):
        sv_, dv, ev, iv, exv, svv, rv, sr, sw = bank[b]
        base = wid * EPW + ch * CH
        pltpu.sync_copy(src_hbm.at[pl.ds(base, CH)], sv_)
        pltpu.sync_copy(dst_hbm.at[pl.ds(base, CH)], dv)
        pltpu.sync_copy(et_hbm.at[pl.ds(base, CH)], ev)
        pltpu.sync_copy(ex_hbm.at[pl.ds(base, CH)], exv)
        for j in range(CH // 16):
            sl = pl.ds(j * 16, 16)
            iv[sl] = ev[sl] * N + sv_[sl]
        pltpu.async_copy(xall_hbm.at[iv], rv, sr)
        pltpu.async_copy(sinv_hbm.at[dv], svv, sw)

    def process(b):
        sv_, dv, ev, iv, exv, svv, rv, sr, sw = bank[b]
        pltpu.make_async_copy(sinv_hbm.at[dv], svv, sw).wait()
        pltpu.make_async_copy(xall_hbm.at[iv], rv, sr).wait()
        for j in range(CH // 16):
            sl = pl.ds(j * 16, 16)
            exv[sl] = exv[sl] * svv[sl]
        for j in range(CH // 16):
            w16 = exv[pl.ds(j * 16, 16)]
            for t in range(16):
                ws = _splat(w16, t)
                for f in range(H // 16):
                    sl = pl.ds(f * 16, 16)
                    rv[j * 16 + t, sl] = rv[j * 16 + t, sl] * ws
        pltpu.sync_copy(rv, acc.at[dv], add=True)

    load_edges(0, 0)

    def chunk(g, _):
        load_edges(2 * g + 1, 1)
        process(0)
        load_edges(2 * g + 2, 0)
        process(1)
        return 0

    lax.fori_loop(0, (NCHUNK - 1) // 2, chunk, 0)
    process(0)
    plsc.subcore_barrier()
    _drain(acc, out_hbm, c, s)


@functools.partial(
    pl.kernel,
    mesh=_mesh,
    out_type=(
        jax.ShapeDtypeStruct((E,), jnp.float32),
        jax.ShapeDtypeStruct((2, NPAD, H), jnp.float32),
    ),
    scratch_types=[
        pltpu.VMEM((CH,), jnp.int32), pltpu.VMEM((CH,), jnp.int32),
        pltpu.VMEM((CH,), jnp.int32), pltpu.VMEM((CH,), jnp.int32),
        pltpu.VMEM((CH,), jnp.int32), pltpu.VMEM((CH,), jnp.int32),
        pltpu.VMEM((CH,), jnp.int32), pltpu.VMEM((CH,), jnp.int32),
        pltpu.VMEM((CH,), jnp.int32), pltpu.VMEM((CH,), jnp.int32),
        pltpu.VMEM((CH,), jnp.float32), pltpu.VMEM((CH,), jnp.float32),
        pltpu.VMEM((CH,), jnp.float32), pltpu.VMEM((CH,), jnp.float32),
        pltpu.VMEM((CH,), jnp.float32), pltpu.VMEM((CH,), jnp.float32),
        pltpu.VMEM((CH,), jnp.float32), pltpu.VMEM((CH,), jnp.float32),
        pltpu.VMEM((CH, H), jnp.float32), pltpu.VMEM((CH, H), jnp.float32),
        pltpu.VMEM_SHARED((NPAD, H), jnp.float32),
        pltpu.VMEM_SHARED((16, H), jnp.float32),
        pltpu.SemaphoreType.DMA, pltpu.SemaphoreType.DMA,
        pltpu.SemaphoreType.DMA, pltpu.SemaphoreType.DMA,
        pltpu.SemaphoreType.DMA, pltpu.SemaphoreType.DMA,
        pltpu.SemaphoreType.DMA, pltpu.SemaphoreType.DMA,
    ],
)
def _sc_logits_fused(ones_hbm, qflat_hbm, kflat_hbm, bflat_hbm, src_hbm,
                     dst_hbm, et_hbm, ex_hbm, sp_hbm,
                     src0, src1, dst0, dst1, et0, et1, qix0, qix1, kix0, kix1,
                     q0, q1, k0, k1, b0, b1, ex0, ex1, rows0, rows1, acc,
                     ones_spm, smq0, smq1, smk0, smk1, smb0, smb1, smr0, smr1):
    c = lax.axis_index("c")
    s = lax.axis_index("s")
    wid = c * 16 + s
    _zero_rows128(rows0)
    _init_acc(rows0, acc, s)

    @pl.when(s == 0)
    def _():
        pltpu.sync_copy(ones_hbm, ones_spm)

    plsc.subcore_barrier()

    bank = ((src0, dst0, et0, qix0, kix0, q0, k0, b0, ex0, rows0,
             smq0, smk0, smb0, smr0),
            (src1, dst1, et1, qix1, kix1, q1, k1, b1, ex1, rows1,
             smq1, smk1, smb1, smr1))

    def load_edges(ch, b):
        sv, dv, ev, qi, ki, qv, kv, bv, exv, rv, smq, smk, smb, smr = bank[b]
        base = wid * EPW + ch * CH
        pltpu.sync_copy(src_hbm.at[pl.ds(base, CH)], sv)
        pltpu.sync_copy(dst_hbm.at[pl.ds(base, CH)], dv)
        pltpu.sync_copy(et_hbm.at[pl.ds(base, CH)], ev)
        for j in range(CH // 16):
            sl = pl.ds(j * 16, 16)
            qi[sl] = dv[sl] * 16 + ev[sl]
            ki[sl] = sv[sl] * 16 + ev[sl]
        pltpu.async_copy(qflat_hbm.at[qi], qv, smq)
        pltpu.async_copy(kflat_hbm.at[ki], kv, smk)
        pltpu.async_copy(bflat_hbm.at[dv], bv, smb)
        pltpu.async_copy(ones_spm.at[ev], rv, smr)

    def process(ch, b):
        sv, dv, ev, qi, ki, qv, kv, bv, exv, rv, smq, smk, smb, smr = bank[b]
        base = wid * EPW + ch * CH
        pltpu.make_async_copy(qflat_hbm.at[qi], qv, smq).wait()
        pltpu.make_async_copy(kflat_hbm.at[ki], kv, smk).wait()
        pltpu.make_async_copy(bflat_hbm.at[dv], bv, smb).wait()
        pltpu.make_async_copy(ones_spm.at[ev], rv, smr).wait()
        for j in range(CH // 16):
            sl = pl.ds(j * 16, 16)
            l = qv[sl] + kv[sl]
            l = jnp.where(l >= 0.0, l, 0.2 * l)
            exv[sl] = jnp.exp(l - bv[sl])
        for j in range(CH // 16):
            w16 = exv[pl.ds(j * 16, 16)]
            for t in range(16):
                ws = _splat(w16, t)
                for f in range(H // 16):
                    sl = pl.ds(f * 16, 16)
                    rv[j * 16 + t, sl] = rv[j * 16 + t, sl] * ws
        pltpu.sync_copy(rv, acc.at[dv], add=True)
        pltpu.sync_copy(exv, ex_hbm.at[pl.ds(base, CH)])

    load_edges(0, 0)

    def chunk(g, _):
        load_edges(2 * g + 1, 1)
        process(2 * g, 0)
        load_edges(2 * g + 2, 0)
        process(2 * g + 1, 1)
        return 0

    lax.fori_loop(0, (NCHUNK - 1) // 2, chunk, 0)
    process(NCHUNK - 1, 0)
    plsc.subcore_barrier()
    _drain(acc, sp_hbm, c, s)


def _xall_body(x_ref, w_ref, o_ref):
    o_ref[0] = jnp.dot(x_ref[...], w_ref[0], preferred_element_type=jnp.float32)


def _xall(h, W):
    out = pl.pallas_call(
        _xall_body,
        grid=(R,),
        in_specs=[
            pl.BlockSpec((N, F), lambda r: (0, 0)),
            pl.BlockSpec((1, F, H), lambda r: (r, 0, 0)),
        ],
        out_specs=pl.BlockSpec((1, N, H), lambda r: (r, 0, 0)),
        out_shape=jax.ShapeDtypeStruct((R, N, H), jnp.float32),
    )(h, W)
    return out.reshape(R * N, H)


def _inv_body(p_ref, o_ref):
    cnt = p_ref[0, :, :16] + p_ref[1, :, :16]
    o_ref[...] = 1.0 / jnp.maximum(cnt, 1.0)


def _inv_table(parts):
    return pl.pallas_call(
        _inv_body,
        out_shape=jax.ShapeDtypeStruct((NPAD, 16), jnp.float32),
    )(parts)


def _epi_rgcn_body(p_ref, h_ref, root_ref, b_ref, g_ref, be_ref, o_ref):
    agg = p_ref[0, :N, :] + p_ref[1, :N, :]
    agg = agg + jnp.dot(h_ref[...], root_ref[...],
                        preferred_element_type=jnp.float32) + b_ref[...]
    mu = jnp.mean(agg, axis=0, keepdims=True)
    var = jnp.mean((agg - mu) ** 2, axis=0, keepdims=True)
    y = (agg - mu) / jnp.sqrt(var + 1e-5) * g_ref[...] + be_ref[...]
    o_ref[...] = jnp.maximum(y, 0.0)


def _epi_rgcn(parts, h, Root, b, g, be):
    return pl.pallas_call(
        _epi_rgcn_body,
        out_shape=jax.ShapeDtypeStruct((N, H), jnp.float32),
    )(parts, h, Root, b.reshape(1, H), g.reshape(1, H), be.reshape(1, H))


def _prep3_body(h_ref, w3_ref, aq_ref, ak_ref, q_ref, k_ref, b_ref):
    h = h_ref[...]
    vq = jnp.einsum("rhk,rk->rh", w3_ref[...], aq_ref[...],
                    preferred_element_type=jnp.float32)
    vk = jnp.einsum("rhk,rk->rh", w3_ref[...], ak_ref[...],
                    preferred_element_type=jnp.float32)
    pad = jnp.zeros((16 - R, H), jnp.float32)
    vq16 = jnp.concatenate([vq, pad], axis=0)
    vk16 = jnp.concatenate([vk, pad], axis=0)
    q = jnp.dot(h, vq16.T, preferred_element_type=jnp.float32)
    k = jnp.dot(h, vk16.T, preferred_element_type=jnp.float32)
    lane = lax.broadcasted_iota(jnp.int32, (1, 16), 1)
    valid = lane < R
    kmax = jnp.max(jnp.where(valid, k, -jnp.inf), axis=0, keepdims=True)
    bpre = jnp.max(jnp.where(valid, q + kmax, -jnp.inf), axis=1, keepdims=True)
    bnd = jnp.where(bpre >= 0.0, bpre, 0.2 * bpre)
    q_ref[...] = jnp.where(valid, q, 0.0)
    k_ref[...] = jnp.where(valid, k, 0.0)
    b_ref[...] = bnd


def _prep3(h, W3, aq, ak):
    return pl.pallas_call(
        _prep3_body,
        out_shape=(
            jax.ShapeDtypeStruct((N, 16), jnp.float32),
            jax.ShapeDtypeStruct((N, 16), jnp.float32),
            jax.ShapeDtypeStruct((N, 1), jnp.float32),
        ),
    )(h, W3, aq, ak)


def _sinv_body(p_ref, o_ref):
    stot = p_ref[0, :, :1] + p_ref[1, :, :1]
    o_ref[...] = 1.0 / jnp.maximum(stot, 1e-16)


def _sinv_table(parts):
    return pl.pallas_call(
        _sinv_body,
        out_shape=jax.ShapeDtypeStruct((NPAD, 1), jnp.float32),
    )(parts)


def _epi3_pool_body(p_ref, b_ref, g_ref, be_ref, batch_ref, wl_ref, bl_ref,
                    o_ref):
    agg = p_ref[0, :N, :] + p_ref[1, :N, :] + b_ref[...]
    mu = jnp.mean(agg, axis=0, keepdims=True)
    var = jnp.mean((agg - mu) ** 2, axis=0, keepdims=True)
    y = (agg - mu) / jnp.sqrt(var + 1e-5) * g_ref[...] + be_ref[...]
    h = jnp.maximum(y, 0.0)
    gid = lax.broadcasted_iota(jnp.int32, (G, N), 0)
    P = (batch_ref[...] == gid).astype(jnp.float32)
    cnt = jnp.sum(P, axis=1, keepdims=True)
    pooled = jnp.dot(P, h, preferred_element_type=jnp.float32)
    pooled = pooled / jnp.maximum(cnt, 1.0)
    o_ref[...] = jnp.dot(pooled, wl_ref[...],
                         preferred_element_type=jnp.float32) + bl_ref[...]


def _epi3_pool(parts, b3, g3, be3, batch, Wl, bl):
    return pl.pallas_call(
        _epi3_pool_body,
        out_shape=jax.ShapeDtypeStruct((G, C), jnp.float32),
    )(parts, b3.reshape(1, H), g3.reshape(1, H), be3.reshape(1, H),
      batch.reshape(1, N), Wl, bl.reshape(1, C))


def kernel(x, edge_index, edge_type, batch, W1, Root1, b1, g1, be1,
           W2, Root2, b2, g2, be2, W3, aq, ak, b3, g3, be3, Wl, bl):
    src = edge_index[0]
    dst = edge_index[1]
    et = edge_type
    eye128 = jnp.concatenate(
        [jnp.eye(16, dtype=jnp.float32),
         jnp.zeros((16, H - 16), jnp.float32)], axis=1)
    ones128 = jnp.ones((16, H), jnp.float32)

    cnt_parts = _sc_cnt(eye128, dst, et)
    wflat = _inv_table(cnt_parts).reshape(NPAD * 16)

    xall1 = _xall(x, W1)
    p1 = _sc_agg_rgcn(xall1, wflat, src, dst, et)
    h1 = _epi_rgcn(p1, x, Root1, b1, g1, be1)

    xall2 = _xall(h1, W2)
    p2 = _sc_agg_rgcn(xall2, wflat, src, dst, et)
    h2 = _epi_rgcn(p2, h1, Root2, b2, g2, be2)

    q16, k16, bnd = _prep3(h2, W3, aq, ak)
    qflat = q16.reshape(N * 16)
    kflat = k16.reshape(N * 16)
    bflat = bnd.reshape(N)
    ex, s_parts = _sc_logits_fused(ones128, qflat, kflat, bflat, src, dst, et)
    sinv = _sinv_table(s_parts).reshape(NPAD)
    xall3 = _xall(h2, W3)
    p3 = _sc_agg_rgat(xall3, sinv, src, dst, et, ex)
    return _epi3_pool(p3, b3, g3, be3, batch, Wl, bl)
